# baseline (speedup 1.0000x reference)
import functools

import jax
import jax.numpy as jnp
from jax.experimental import pallas as pl
from jax.experimental.pallas import tpu as pltpu

_NXC = 2
_NSC = 2


def _gru_chunk_kernel(first, c_base, nsc,
                      emb_ref, w_ih_ref, w_hh_ref, bias_ref, b_hn_ref,
                      *rest):
    if first:
        out_ref, hid_ref, gi_ref, h_ref = rest
        h_in_ref = None
    else:
        h_in_ref, _obuf_ref, out_ref, hid_ref, gi_ref, h_ref = rest

    Ts, B, H = emb_ref.shape
    H2 = 2 * H
    s = pl.program_id(0)

    gi = jax.lax.dot_general(
        emb_ref[...], w_ih_ref[...],
        dimension_numbers=(((2,), (0,)), ((), ())),
        preferred_element_type=jnp.float32)
    gi_ref[...] = gi + bias_ref[...]

    @pl.when(s == 0)
    def _init():
        if first:
            h_ref[...] = jnp.zeros_like(h_ref)
        else:
            h_ref[...] = h_in_ref[...]

    b_hn = jnp.broadcast_to(b_hn_ref[...], (B, H))
    h = h_ref[...]

    for t in range(Ts):
        gi_t = gi_ref[t]
        h_b = h.astype(jnp.bfloat16)

        gh_rz = jnp.dot(h_b, w_hh_ref[:, 0:H2],
                        preferred_element_type=jnp.float32)
        r = jax.nn.sigmoid(gi_t[:, 0:H] + gh_rz[:, 0:H])
        z = jax.nn.sigmoid(gi_t[:, H:H2] + gh_rz[:, H:H2])

        gh_n = jnp.dot(h_b, w_hh_ref[:, H2:],
                       preferred_element_type=jnp.float32)
        n = jnp.tanh(gi_t[:, H2:] + r * (gh_n + b_hn))

        h = n + z * (h - n)
        out_ref[:, t, :] = h

    h_ref[...] = h
    hid_ref[...] = h


def kernel(x_ids, emb_table, w_ih, w_hh, b_ih, b_hh):
    B, T = x_ids.shape
    H = emb_table.shape[1]
    nxc = _NXC if T % (_NXC * _NSC) == 0 else 1
    nsc = _NSC if T % (_NXC * _NSC) == 0 else 1
    tc = T // nxc
    ts = tc // nsc

    x_tm = x_ids.T
    w_ih_t = w_ih.T.astype(jnp.bfloat16)
    w_hh_t = w_hh.T.astype(jnp.bfloat16)
    b_rz = jnp.concatenate([b_hh[:2 * H], jnp.zeros((H,), b_hh.dtype)])
    bias = (b_ih + b_rz).reshape(1, 3 * H)
    b_hn = b_hh[2 * H:].reshape(1, H)

    out_buf = None
    hid = None
    for c in range(nxc):
        first = c == 0
        emb_c = emb_table[x_tm[c * tc:(c + 1) * tc]].astype(jnp.bfloat16)

        in_specs = [
            pl.BlockSpec((ts, B, H), lambda s: (s, 0, 0)),
            pl.BlockSpec((H, 3 * H), lambda s: (0, 0)),
            pl.BlockSpec((H, 3 * H), lambda s: (0, 0)),
            pl.BlockSpec((1, 3 * H), lambda s: (0, 0)),
            pl.BlockSpec((1, H), lambda s: (0, 0)),
        ]
        args = [emb_c, w_ih_t, w_hh_t, bias, b_hn]
        aliases = {}
        if not first:
            in_specs.append(pl.BlockSpec((B, H), lambda s: (0, 0)))
            in_specs.append(pl.BlockSpec(memory_space=pl.ANY))
            args.extend([hid, out_buf])
            aliases = {6: 0}

        c2 = c * nsc
        out_buf, hid = pl.pallas_call(
            functools.partial(_gru_chunk_kernel, first, c2, nsc),
            out_shape=(
                jax.ShapeDtypeStruct((B, T, H), jnp.float32),
                jax.ShapeDtypeStruct((B, H), jnp.float32),
            ),
            grid=(nsc,),
            in_specs=in_specs,
            out_specs=(
                pl.BlockSpec((B, ts, H),
                             lambda s, c2=c2: (0, c2 + s, 0)),
                pl.BlockSpec((B, H), lambda s: (0, 0)),
            ),
            scratch_shapes=[
                pltpu.VMEM((ts, B, 3 * H), jnp.float32),
                pltpu.VMEM((B, H), jnp.float32),
            ],
            input_output_aliases=aliases,
            compiler_params=pltpu.CompilerParams(
                dimension_semantics=("arbitrary",)),
        )(*args)

    return out_buf, hid.reshape(1, B, H)


# speedup vs baseline: 1.3721x; 1.3721x over previous
import jax
import jax.numpy as jnp
from jax.experimental import pallas as pl
from jax.experimental.pallas import tpu as pltpu

_NC = 4


def _gru_fused_kernel(emb_ref, w_ih_ref, w_hh_ref, bias_ref, b_hn_ref,
                      out_ref, hid_ref, gi_ref, h_ref):
    Tc, B, H = emb_ref.shape
    H2 = 2 * H
    c = pl.program_id(0)

    gi = jax.lax.dot_general(
        emb_ref[...].astype(jnp.bfloat16), w_ih_ref[...],
        dimension_numbers=(((2,), (0,)), ((), ())),
        preferred_element_type=jnp.float32)
    gi_ref[...] = gi + bias_ref[...]

    @pl.when(c == 0)
    def _init():
        h_ref[...] = jnp.zeros_like(h_ref)

    b_hn = jnp.broadcast_to(b_hn_ref[...], (B, H))
    h = h_ref[...]

    for t in range(Tc):
        gi_t = gi_ref[t]
        h_b = h.astype(jnp.bfloat16)

        gh_rz = jnp.dot(h_b, w_hh_ref[:, 0:H2],
                        preferred_element_type=jnp.float32)
        r = jax.nn.sigmoid(gi_t[:, 0:H] + gh_rz[:, 0:H])
        z = jax.nn.sigmoid(gi_t[:, H:H2] + gh_rz[:, H:H2])

        gh_n = jnp.dot(h_b, w_hh_ref[:, H2:],
                       preferred_element_type=jnp.float32)
        n = jnp.tanh(gi_t[:, H2:] + r * (gh_n + b_hn))

        h = n + z * (h - n)
        out_ref[:, t, :] = h

    h_ref[...] = h
    hid_ref[...] = h


def kernel(x_ids, emb_table, w_ih, w_hh, b_ih, b_hh):
    B, T = x_ids.shape
    H = emb_table.shape[1]
    nc = _NC if T % _NC == 0 else 1
    tc = T // nc

    embedded_tm = emb_table[x_ids.T]

    w_ih_t = w_ih.T.astype(jnp.bfloat16)
    w_hh_t = w_hh.T.astype(jnp.bfloat16)
    b_rz = jnp.concatenate([b_hh[:2 * H], jnp.zeros((H,), b_hh.dtype)])
    bias = (b_ih + b_rz).reshape(1, 3 * H)
    b_hn = b_hh[2 * H:].reshape(1, H)

    output, hidden = pl.pallas_call(
        _gru_fused_kernel,
        out_shape=(
            jax.ShapeDtypeStruct((B, T, H), jnp.float32),
            jax.ShapeDtypeStruct((B, H), jnp.float32),
        ),
        grid=(nc,),
        in_specs=[
            pl.BlockSpec((tc, B, H), lambda c: (c, 0, 0)),
            pl.BlockSpec((H, 3 * H), lambda c: (0, 0)),
            pl.BlockSpec((H, 3 * H), lambda c: (0, 0)),
            pl.BlockSpec((1, 3 * H), lambda c: (0, 0)),
            pl.BlockSpec((1, H), lambda c: (0, 0)),
        ],
        out_specs=(
            pl.BlockSpec((B, tc, H), lambda c: (0, c, 0)),
            pl.BlockSpec((B, H), lambda c: (0, 0)),
        ),
        scratch_shapes=[
            pltpu.VMEM((tc, B, 3 * H), jnp.float32),
            pltpu.VMEM((B, H), jnp.float32),
        ],
        compiler_params=pltpu.CompilerParams(
            dimension_semantics=("arbitrary",)),
    )(embedded_tm, w_ih_t, w_hh_t, bias, b_hn)

    return output, hidden.reshape(1, B, H)
